# Initial kernel scaffold; baseline (speedup 1.0000x reference)
import jax
import jax.numpy as jnp
from jax.experimental import pallas as pl


def kernel(pair_emb, protein_length, token_length, W1, b1, g1, be1, Wa1, ba1, Wa2, ba2, Wo, bo, g2, be2):
    raise NotImplementedError("write your pallas kernel here")


# speedup vs baseline: 5.7211x; 5.7211x over previous
import functools

import jax
import jax.numpy as jnp
from jax.experimental import pallas as pl
from jax.experimental.pallas import tpu as pltpu

_T = 384
_D = 128
_H = 256
_HH = 128
_BR = 16
_NEG = -1e30
_INV_SQRT2 = 0.7071067811865476


def _gelu_exact(x):
    return 0.5 * x * (1.0 + jax.lax.erf(x * _INV_SQRT2))


def _fused_kernel(plen_ref, tlen_ref, pair_ref,
                  W1_ref, b1_ref, g1_ref, be1_ref,
                  Wa1_ref, ba1_ref, wa2_ref, ba2_ref,
                  Wo_ref, bo_ref, g2_ref, be2_ref,
                  out_ref, m_ref, s_ref, p_ref):
    b = pl.program_id(0)
    i = pl.program_id(1)
    nb = pl.num_programs(1)

    @pl.when(i == 0)
    def _init():
        m_ref[0] = _NEG
        s_ref[0] = 0.0
        p_ref[...] = jnp.zeros_like(p_ref)

    x_in = pair_ref[...].reshape(_BR * _T, _D)
    y = jnp.dot(x_in, W1_ref[...], preferred_element_type=jnp.float32) + b1_ref[...]
    mu = jnp.mean(y, axis=-1, keepdims=True)
    var = jnp.mean((y - mu) * (y - mu), axis=-1, keepdims=True)
    yn = (y - mu) * jax.lax.rsqrt(var + 1e-5) * g1_ref[...] + be1_ref[...]
    x = _gelu_exact(yn)

    a = jnp.tanh(jnp.dot(x, Wa1_ref[...], preferred_element_type=jnp.float32) + ba1_ref[...])
    attn = jnp.sum(a * wa2_ref[...], axis=-1, keepdims=True) + ba2_ref[0, 0]

    P = plen_ref[b]
    L = tlen_ref[b]
    k = jax.lax.broadcasted_iota(jnp.int32, (_BR * _T, 1), 0)
    g = jax.lax.shift_right_logical(
        jax.lax.shift_right_logical(k, 7) * 21846, 16)
    ri = i * _BR + g
    ci = k - g * _T
    pm_r = ri < P
    lm_r = jnp.logical_and(ri >= P, ri < L)
    pm_c = ci < P
    lm_c = jnp.logical_and(ci >= P, ci < L)
    inter = jnp.logical_or(jnp.logical_and(pm_r, lm_c),
                           jnp.logical_and(lm_r, pm_c))

    sc = jnp.where(inter, attn, _NEG)
    m_old = m_ref[0]
    m_new = jnp.maximum(m_old, jnp.max(sc))
    w = jnp.where(inter, jnp.exp(sc - m_new), 0.0)
    alpha = jnp.exp(m_old - m_new)
    m_ref[0] = m_new
    s_ref[0] = s_ref[0] * alpha + jnp.sum(w)
    p_ref[...] = p_ref[...] * alpha + jnp.sum(x * w, axis=0, keepdims=True)

    @pl.when(i == nb - 1)
    def _finish():
        pooled = p_ref[...] / jnp.maximum(s_ref[0], 1e-30)
        z = jnp.dot(pooled, Wo_ref[...], preferred_element_type=jnp.float32) + bo_ref[...]
        mu2 = jnp.mean(z, axis=-1, keepdims=True)
        var2 = jnp.mean((z - mu2) * (z - mu2), axis=-1, keepdims=True)
        zn = (z - mu2) * jax.lax.rsqrt(var2 + 1e-5) * g2_ref[...] + be2_ref[...]
        out_ref[...] = _gelu_exact(zn).reshape(out_ref.shape)


@functools.partial(jax.jit, static_argnames=())
def kernel(pair_emb, protein_length, token_length, W1, b1, g1, be1,
           Wa1, ba1, Wa2, ba2, Wo, bo, g2, be2):
    B, T, _, D = pair_emb.shape
    H = W1.shape[1]
    nb = T // _BR

    row = lambda v: v.reshape(1, -1)
    wa2_row = Wa2.reshape(1, _HH)

    grid = (B, nb)
    out = pl.pallas_call(
        _fused_kernel,
        grid=grid,
        in_specs=[
            pl.BlockSpec(memory_space=pltpu.SMEM),
            pl.BlockSpec(memory_space=pltpu.SMEM),
            pl.BlockSpec((1, _BR, T, D), lambda b, i: (b, i, 0, 0)),
            pl.BlockSpec((D, H), lambda b, i: (0, 0)),
            pl.BlockSpec((1, H), lambda b, i: (0, 0)),
            pl.BlockSpec((1, H), lambda b, i: (0, 0)),
            pl.BlockSpec((1, H), lambda b, i: (0, 0)),
            pl.BlockSpec((H, _HH), lambda b, i: (0, 0)),
            pl.BlockSpec((1, _HH), lambda b, i: (0, 0)),
            pl.BlockSpec((1, _HH), lambda b, i: (0, 0)),
            pl.BlockSpec((1, 1), lambda b, i: (0, 0)),
            pl.BlockSpec((H, H), lambda b, i: (0, 0)),
            pl.BlockSpec((1, H), lambda b, i: (0, 0)),
            pl.BlockSpec((1, H), lambda b, i: (0, 0)),
            pl.BlockSpec((1, H), lambda b, i: (0, 0)),
        ],
        out_specs=pl.BlockSpec((1, 1, H), lambda b, i: (b, 0, 0)),
        out_shape=jax.ShapeDtypeStruct((B, 1, H), jnp.float32),
        scratch_shapes=[
            pltpu.SMEM((1,), jnp.float32),
            pltpu.SMEM((1,), jnp.float32),
            pltpu.VMEM((1, H), jnp.float32),
        ],
        compiler_params=pltpu.CompilerParams(
            dimension_semantics=("arbitrary", "arbitrary"),
        ),
    )(protein_length, token_length, pair_emb,
      W1, row(b1), row(g1), row(be1),
      Wa1, row(ba1), wa2_row, ba2.reshape(1, 1),
      Wo, row(bo), row(g2), row(be2))
    return out.reshape(B, H)
